# initial kernel scaffold (unmeasured)
import jax
import jax.numpy as jnp
from jax import lax
from jax.experimental import pallas as pl
from jax.experimental.pallas import tpu as pltpu

N_DEV = 16
SQ = 1024
SKV = 1024
HQ_PER = 8
DH = 128
CHUNK = SQ // N_DEV
SCALE = 0.08838834764831843


def _compute_body(x_ref, wq_ref, k_ref, v_ref, wo_ref, out_ref):
    xb = x_ref[...].astype(jnp.bfloat16)
    wqb = wq_ref[...].astype(jnp.bfloat16)
    q = lax.dot_general(
        xb, wqb, (((1,), (0,)), ((), ())), preferred_element_type=jnp.float32
    ).astype(jnp.bfloat16)

    qi = lax.broadcasted_iota(jnp.int32, (SQ, SKV), 0)
    ki = lax.broadcasted_iota(jnp.int32, (SQ, SKV), 1)
    mask = (jnp.abs(qi - ki) <= 128) | (ki < 32) | (qi < 32)

    ctx_parts = []
    for h in range(HQ_PER):
        qh = q[:, h * DH:(h + 1) * DH]
        kh = k_ref[:, h, :].astype(jnp.bfloat16)
        vh = v_ref[:, h, :].astype(jnp.bfloat16)
        s = lax.dot_general(
            qh, kh, (((1,), (1,)), ((), ())), preferred_element_type=jnp.float32
        ) * SCALE
        s = jnp.where(mask, s, -1e9)
        m = jnp.max(s, axis=1, keepdims=True)
        w = jnp.exp(s - m)
        w = w / jnp.sum(w, axis=1, keepdims=True)
        ctx_parts.append(
            lax.dot_general(
                w.astype(jnp.bfloat16), vh, (((1,), (0,)), ((), ())),
                preferred_element_type=jnp.float32,
            )
        )
    ctx = jnp.concatenate(ctx_parts, axis=1).astype(jnp.bfloat16)
    wob = wo_ref[...].astype(jnp.bfloat16)
    out_ref[...] = lax.dot_general(
        ctx, wob, (((1,), (0,)), ((), ())), preferred_element_type=jnp.float32
    )


def _allreduce_body(p_ref, out_ref, sbuf, rs_comm, ag_comm,
                    rs_send, rs_recv, ag_send, ag_recv):
    my = lax.axis_index("i")
    right = lax.rem(my + 1, N_DEV)

    def rows(c):
        return pl.ds(c * CHUNK, CHUNK)

    sbuf[...] = p_ref[rows(my), :]
    for s in range(N_DEV - 1):
        rdma = pltpu.make_async_remote_copy(
            src_ref=sbuf,
            dst_ref=rs_comm.at[s],
            send_sem=rs_send.at[s],
            recv_sem=rs_recv.at[s],
            device_id=(right,),
            device_id_type=pl.DeviceIdType.MESH,
        )
        rdma.start()
        rdma.wait()
        c = lax.rem(my - s - 1 + N_DEV, N_DEV)
        nxt = rs_comm[s] + p_ref[rows(c), :]
        if s < N_DEV - 2:
            sbuf[...] = nxt
        else:
            out_ref[rows(c), :] = nxt
            sbuf[...] = nxt

    for t in range(N_DEV - 1):
        src = sbuf if t == 0 else ag_comm.at[t - 1]
        rdma = pltpu.make_async_remote_copy(
            src_ref=src,
            dst_ref=ag_comm.at[t],
            send_sem=ag_send.at[t],
            recv_sem=ag_recv.at[t],
            device_id=(right,),
            device_id_type=pl.DeviceIdType.MESH,
        )
        rdma.start()
        rdma.wait()
        c = lax.rem(my - t + N_DEV, N_DEV)
        out_ref[rows(c), :] = ag_comm[t]


def kernel(x, Wq, K_ext, V_ext, Wo):
    i = lax.axis_index("i")
    x2 = x[0]
    K = lax.dynamic_slice_in_dim(K_ext[0], i * HQ_PER, HQ_PER, axis=1)
    V = lax.dynamic_slice_in_dim(V_ext[0], i * HQ_PER, HQ_PER, axis=1)

    partial = pl.pallas_call(
        _compute_body,
        out_shape=jax.ShapeDtypeStruct((SQ, 1024), jnp.float32),
        in_specs=[pl.BlockSpec(memory_space=pltpu.VMEM)] * 5,
        out_specs=pl.BlockSpec(memory_space=pltpu.VMEM),
    )(x2, Wq, K, V, Wo)

    out = pl.pallas_call(
        _allreduce_body,
        out_shape=jax.ShapeDtypeStruct((SQ, 1024), jnp.float32),
        in_specs=[pl.BlockSpec(memory_space=pltpu.VMEM)],
        out_specs=pl.BlockSpec(memory_space=pltpu.VMEM),
        scratch_shapes=[
            pltpu.VMEM((CHUNK, 1024), jnp.float32),
            pltpu.VMEM((N_DEV - 1, CHUNK, 1024), jnp.float32),
            pltpu.VMEM((N_DEV - 1, CHUNK, 1024), jnp.float32),
            pltpu.SemaphoreType.DMA((N_DEV - 1,)),
            pltpu.SemaphoreType.DMA((N_DEV - 1,)),
            pltpu.SemaphoreType.DMA((N_DEV - 1,)),
            pltpu.SemaphoreType.DMA((N_DEV - 1,)),
        ],
        compiler_params=pltpu.CompilerParams(collective_id=0),
    )(partial)

    return out[None]


# baseline (device time: 185824 ns/iter reference)
import jax
import jax.numpy as jnp
from jax import lax
from jax.experimental import pallas as pl
from jax.experimental.pallas import tpu as pltpu

N_DEV = 16
SQ = 1024
SKV = 1024
HQ_PER = 8
DH = 128
CHUNK = SQ // N_DEV
SCALE = 0.08838834764831843


def _compute_body(x_ref, wq_ref, k_ref, v_ref, wo_ref, out_ref):
    xb = x_ref[...].astype(jnp.bfloat16)
    wqb = wq_ref[...].astype(jnp.bfloat16)
    q = lax.dot_general(
        xb, wqb, (((1,), (0,)), ((), ())), preferred_element_type=jnp.float32
    ).astype(jnp.bfloat16)

    qi = lax.broadcasted_iota(jnp.int32, (SQ, SKV), 0)
    ki = lax.broadcasted_iota(jnp.int32, (SQ, SKV), 1)
    mask = (jnp.abs(qi - ki) <= 128) | (ki < 32) | (qi < 32)

    ctx_parts = []
    for h in range(HQ_PER):
        qh = q[:, h * DH:(h + 1) * DH]
        kh = k_ref[:, h, :].astype(jnp.bfloat16)
        vh = v_ref[:, h, :].astype(jnp.bfloat16)
        s = lax.dot_general(
            qh, kh, (((1,), (1,)), ((), ())), preferred_element_type=jnp.float32
        ) * SCALE
        s = jnp.where(mask, s, -1e9)
        m = jnp.max(s, axis=1, keepdims=True)
        w = jnp.exp(s - m)
        w = w / jnp.sum(w, axis=1, keepdims=True)
        ctx_parts.append(
            lax.dot_general(
                w.astype(jnp.bfloat16), vh, (((1,), (0,)), ((), ())),
                preferred_element_type=jnp.float32,
            )
        )
    ctx = jnp.concatenate(ctx_parts, axis=1).astype(jnp.bfloat16)
    wob = wo_ref[...].astype(jnp.bfloat16)
    out_ref[...] = lax.dot_general(
        ctx, wob, (((1,), (0,)), ((), ())), preferred_element_type=jnp.float32
    )


def _allreduce_body(p_ref, out_ref, sbuf, rs_comm, ag_comm,
                    rs_send, rs_recv, ag_send, ag_recv):
    my = lax.axis_index("i")
    right = lax.rem(my + 1, N_DEV)

    def rows(c):
        return pl.ds(c * CHUNK, CHUNK)

    sbuf[...] = p_ref[rows(my), :]
    for s in range(N_DEV - 1):
        rdma = pltpu.make_async_remote_copy(
            src_ref=sbuf,
            dst_ref=rs_comm.at[s],
            send_sem=rs_send.at[s],
            recv_sem=rs_recv.at[s],
            device_id=(right,),
            device_id_type=pl.DeviceIdType.MESH,
        )
        rdma.start()
        rdma.wait()
        c = lax.rem(my - s - 1 + N_DEV, N_DEV)
        nxt = rs_comm[s] + p_ref[rows(c), :]
        if s < N_DEV - 2:
            sbuf[...] = nxt
        else:
            out_ref[rows(c), :] = nxt
            sbuf[...] = nxt

    for t in range(N_DEV - 1):
        src = sbuf if t == 0 else ag_comm.at[t - 1]
        rdma = pltpu.make_async_remote_copy(
            src_ref=src,
            dst_ref=ag_comm.at[t],
            send_sem=ag_send.at[t],
            recv_sem=ag_recv.at[t],
            device_id=(right,),
            device_id_type=pl.DeviceIdType.MESH,
        )
        rdma.start()
        rdma.wait()
        c = lax.rem(my - t + N_DEV, N_DEV)
        out_ref[rows(c), :] = ag_comm[t]


def kernel(x, Wq, K_ext, V_ext, Wo):
    i = lax.axis_index("i")
    x2 = x[0]
    K = lax.dynamic_slice_in_dim(K_ext[0], i * HQ_PER, HQ_PER, axis=1)
    V = lax.dynamic_slice_in_dim(V_ext[0], i * HQ_PER, HQ_PER, axis=1)

    partial = pl.pallas_call(
        _compute_body,
        out_shape=jax.ShapeDtypeStruct((SQ, 1024), jnp.float32),
        in_specs=[pl.BlockSpec(memory_space=pltpu.VMEM)] * 5,
        out_specs=pl.BlockSpec(memory_space=pltpu.VMEM),
    )(x2, Wq, K, V, Wo)

    out = pl.pallas_call(
        _allreduce_body,
        out_shape=jax.ShapeDtypeStruct((SQ, 1024), jnp.float32),
        in_specs=[pl.BlockSpec(memory_space=pltpu.VMEM)],
        out_specs=pl.BlockSpec(memory_space=pltpu.VMEM),
        scratch_shapes=[
            pltpu.VMEM((CHUNK, 1024), jnp.float32),
            pltpu.VMEM((N_DEV - 1, CHUNK, 1024), jnp.float32),
            pltpu.VMEM((N_DEV - 1, CHUNK, 1024), jnp.float32),
            pltpu.SemaphoreType.DMA((N_DEV - 1,)),
            pltpu.SemaphoreType.DMA((N_DEV - 1,)),
            pltpu.SemaphoreType.DMA((N_DEV - 1,)),
            pltpu.SemaphoreType.DMA((N_DEV - 1,)),
        ],
    )(partial)

    return out[None]


# device time: 103383 ns/iter; 1.7974x vs baseline; 1.7974x over previous
import jax
import jax.numpy as jnp
from jax import lax
from jax.experimental import pallas as pl
from jax.experimental.pallas import tpu as pltpu

N_DEV = 16
SQ = 1024
SKV = 1024
HQ_PER = 8
DH = 128
CHUNK = SQ // N_DEV
SCALE = 0.08838834764831843


def _compute_body(x_ref, wq_ref, k_ref, v_ref, wo_ref, out_ref):
    xb = x_ref[...].astype(jnp.bfloat16)
    wqb = wq_ref[...].astype(jnp.bfloat16)
    q = lax.dot_general(
        xb, wqb, (((1,), (0,)), ((), ())), preferred_element_type=jnp.float32
    ).astype(jnp.bfloat16)

    qi = lax.broadcasted_iota(jnp.int32, (SQ, SKV), 0)
    ki = lax.broadcasted_iota(jnp.int32, (SQ, SKV), 1)
    mask = (jnp.abs(qi - ki) <= 128) | (ki < 32) | (qi < 32)

    ctx_parts = []
    for h in range(HQ_PER):
        qh = q[:, h * DH:(h + 1) * DH]
        kh = k_ref[:, h, :].astype(jnp.bfloat16)
        vh = v_ref[:, h, :].astype(jnp.bfloat16)
        s = lax.dot_general(
            qh, kh, (((1,), (1,)), ((), ())), preferred_element_type=jnp.float32
        ) * SCALE
        s = jnp.where(mask, s, -1e9)
        m = jnp.max(s, axis=1, keepdims=True)
        w = jnp.exp(s - m)
        w = w / jnp.sum(w, axis=1, keepdims=True)
        ctx_parts.append(
            lax.dot_general(
                w.astype(jnp.bfloat16), vh, (((1,), (0,)), ((), ())),
                preferred_element_type=jnp.float32,
            )
        )
    ctx = jnp.concatenate(ctx_parts, axis=1).astype(jnp.bfloat16)
    wob = wo_ref[...].astype(jnp.bfloat16)
    out_ref[...] = lax.dot_general(
        ctx, wob, (((1,), (0,)), ((), ())), preferred_element_type=jnp.float32
    )


def _hypercube_partners():
    my = lax.axis_index("i")
    z = my // 4
    j = my % 4
    b = j // 2
    a = (j % 2) ^ b
    h = [a, b, z % 2, z // 2]

    def to_logical(hb):
        jj = (hb[0] ^ hb[1]) + 2 * hb[1]
        return 4 * (2 * hb[3] + hb[2]) + jj

    partners = []
    for k in range(4):
        hb = list(h)
        hb[k] = 1 - hb[k]
        partners.append(to_logical(hb))
    return h, partners


def _allreduce_body(p_ref, out_ref, *s):
    rs_sb, rs_rb = s[0:4], s[4:8]
    ag_sb, ag_rb = s[8:12], s[12:16]
    rs_ssem, rs_rsem, ag_ssem, ag_rsem = s[16:20]

    h, partners = _hypercube_partners()

    out_ref[...] = p_ref[...]

    r = 0
    size = SQ
    for k in range(4):
        half = size // 2
        keep_off = r + h[k] * half
        send_off = r + (1 - h[k]) * half
        rs_sb[k][...] = out_ref[pl.ds(send_off, half), :].astype(jnp.bfloat16)
        rdma = pltpu.make_async_remote_copy(
            src_ref=rs_sb[k],
            dst_ref=rs_rb[k],
            send_sem=rs_ssem.at[k],
            recv_sem=rs_rsem.at[k],
            device_id=(partners[k],),
            device_id_type=pl.DeviceIdType.MESH,
        )
        rdma.start()
        rdma.wait()
        out_ref[pl.ds(keep_off, half), :] = (
            out_ref[pl.ds(keep_off, half), :]
            + rs_rb[k][...].astype(jnp.float32)
        )
        r = keep_off
        size = half

    for k in reversed(range(4)):
        sz2 = SQ >> (k + 1)
        base = r - h[k] * sz2
        other_off = base + (1 - h[k]) * sz2
        ag_sb[k][...] = out_ref[pl.ds(r, sz2), :].astype(jnp.bfloat16)
        rdma = pltpu.make_async_remote_copy(
            src_ref=ag_sb[k],
            dst_ref=ag_rb[k],
            send_sem=ag_ssem.at[k],
            recv_sem=ag_rsem.at[k],
            device_id=(partners[k],),
            device_id_type=pl.DeviceIdType.MESH,
        )
        rdma.start()
        rdma.wait()
        out_ref[pl.ds(other_off, sz2), :] = ag_rb[k][...].astype(jnp.float32)
        r = base


def kernel(x, Wq, K_ext, V_ext, Wo):
    i = lax.axis_index("i")
    x2 = x[0]
    K = lax.dynamic_slice_in_dim(K_ext[0], i * HQ_PER, HQ_PER, axis=1)
    V = lax.dynamic_slice_in_dim(V_ext[0], i * HQ_PER, HQ_PER, axis=1)

    partial = pl.pallas_call(
        _compute_body,
        out_shape=jax.ShapeDtypeStruct((SQ, 1024), jnp.float32),
        in_specs=[pl.BlockSpec(memory_space=pltpu.VMEM)] * 5,
        out_specs=pl.BlockSpec(memory_space=pltpu.VMEM),
    )(x2, Wq, K, V, Wo)

    out = pl.pallas_call(
        _allreduce_body,
        out_shape=jax.ShapeDtypeStruct((SQ, 1024), jnp.float32),
        in_specs=[pl.BlockSpec(memory_space=pltpu.VMEM)],
        out_specs=pl.BlockSpec(memory_space=pltpu.VMEM),
        scratch_shapes=(
            [pltpu.VMEM((SQ >> (k + 1), 1024), jnp.bfloat16) for k in range(4)]
            + [pltpu.VMEM((SQ >> (k + 1), 1024), jnp.bfloat16) for k in range(4)]
            + [pltpu.VMEM((SQ >> (k + 1), 1024), jnp.bfloat16) for k in range(4)]
            + [pltpu.VMEM((SQ >> (k + 1), 1024), jnp.bfloat16) for k in range(4)]
            + [pltpu.SemaphoreType.DMA((4,)) for _ in range(4)]
        ),
    )(partial)

    return out[None]


# device time: 80415 ns/iter; 2.3108x vs baseline; 1.2856x over previous
import jax
import jax.numpy as jnp
from jax import lax
from jax.experimental import pallas as pl
from jax.experimental.pallas import tpu as pltpu

N_DEV = 16
SQ = 1024
SKV = 1024
HQ_PER = 8
DH = 128
CHUNK = SQ // N_DEV
SCALE = 0.08838834764831843


def _compute_body(x_ref, wq_ref, k_ref, v_ref, wo_ref, out_ref):
    xb = x_ref[...].astype(jnp.bfloat16)
    wqb = wq_ref[...].astype(jnp.bfloat16)
    q = lax.dot_general(
        xb, wqb, (((1,), (0,)), ((), ())), preferred_element_type=jnp.float32
    ).astype(jnp.bfloat16)

    qi = lax.broadcasted_iota(jnp.int32, (SQ, SKV), 0)
    ki = lax.broadcasted_iota(jnp.int32, (SQ, SKV), 1)
    mask = (jnp.abs(qi - ki) <= 128) | (ki < 32) | (qi < 32)

    ctx_parts = []
    for h in range(HQ_PER):
        qh = q[:, h * DH:(h + 1) * DH]
        kh = k_ref[:, h, :].astype(jnp.bfloat16)
        vh = v_ref[:, h, :].astype(jnp.bfloat16)
        s = lax.dot_general(
            qh, kh, (((1,), (1,)), ((), ())), preferred_element_type=jnp.float32
        ) * SCALE
        s = jnp.where(mask, s, -1e9)
        m = jnp.max(s, axis=1, keepdims=True)
        w = jnp.exp(s - m)
        w = w / jnp.sum(w, axis=1, keepdims=True)
        ctx_parts.append(
            lax.dot_general(
                w.astype(jnp.bfloat16), vh, (((1,), (0,)), ((), ())),
                preferred_element_type=jnp.float32,
            )
        )
    ctx = jnp.concatenate(ctx_parts, axis=1).astype(jnp.bfloat16)
    wob = wo_ref[...].astype(jnp.bfloat16)
    out_ref[...] = lax.dot_general(
        ctx, wob, (((1,), (0,)), ((), ())), preferred_element_type=jnp.float32
    )


def _hypercube_partners():
    my = lax.axis_index("i")
    z = my // 4
    j = my % 4
    b = j // 2
    a = (j % 2) ^ b
    h = [a, b, z % 2, z // 2]

    def to_logical(hb):
        jj = (hb[0] ^ hb[1]) + 2 * hb[1]
        return 4 * (2 * hb[3] + hb[2]) + jj

    partners = []
    for k in range(4):
        hb = list(h)
        hb[k] = 1 - hb[k]
        partners.append(to_logical(hb))
    return h, partners


ORDERS = ((0, 1, 2, 3), (1, 2, 0, 3))
HALF_ROWS = SQ // 2


def _rs_rows(t):
    return HALF_ROWS >> (t + 1)


def _ag_rows(u):
    return 32 << u


def _allreduce_body(p_ref, out_ref, *s):
    rs_s = (s[0:4], s[16:20])
    rs_r = (s[4:8], s[20:24])
    ag_s = (s[8:12], s[24:28])
    ag_r = (s[12:16], s[28:32])
    sems = s[32:40]
    rs_ss = (sems[0], sems[4])
    rs_rs = (sems[1], sems[5])
    ag_ss = (sems[2], sems[6])
    ag_rs = (sems[3], sems[7])

    h, partners = _hypercube_partners()

    barrier_sem = pltpu.get_barrier_semaphore()
    for d in range(4):
        pl.semaphore_signal(
            barrier_sem, inc=1,
            device_id=(partners[d],), device_id_type=pl.DeviceIdType.MESH,
        )
    pl.semaphore_wait(barrier_sem, 4)

    out_ref[...] = p_ref[...]

    r = [0, HALF_ROWS]
    size = [HALF_ROWS, HALF_ROWS]

    for t in range(4):
        started = []
        for bf in range(2):
            d = ORDERS[bf][t]
            half = size[bf] // 2
            keep = r[bf] + h[d] * half
            send = r[bf] + (1 - h[d]) * half
            rs_s[bf][t][...] = out_ref[pl.ds(send, half), :].astype(jnp.bfloat16)
            rdma = pltpu.make_async_remote_copy(
                src_ref=rs_s[bf][t],
                dst_ref=rs_r[bf][t],
                send_sem=rs_ss[bf].at[t],
                recv_sem=rs_rs[bf].at[t],
                device_id=(partners[d],),
                device_id_type=pl.DeviceIdType.MESH,
            )
            rdma.start()
            started.append((rdma, keep, half))
            r[bf] = keep
            size[bf] = half
        for bf in range(2):
            rdma, keep, half = started[bf]
            rdma.wait()
            out_ref[pl.ds(keep, half), :] = (
                out_ref[pl.ds(keep, half), :]
                + rs_r[bf][t][...].astype(jnp.float32)
            )

    for u in range(4):
        started = []
        for bf in range(2):
            d = ORDERS[bf][3 - u]
            sz = size[bf]
            base = r[bf] - h[d] * sz
            other = base + (1 - h[d]) * sz
            ag_s[bf][u][...] = out_ref[pl.ds(r[bf], sz), :].astype(jnp.bfloat16)
            rdma = pltpu.make_async_remote_copy(
                src_ref=ag_s[bf][u],
                dst_ref=ag_r[bf][u],
                send_sem=ag_ss[bf].at[u],
                recv_sem=ag_rs[bf].at[u],
                device_id=(partners[d],),
                device_id_type=pl.DeviceIdType.MESH,
            )
            rdma.start()
            started.append((rdma, other, sz))
            r[bf] = base
            size[bf] = sz * 2
        for bf in range(2):
            rdma, other, sz = started[bf]
            rdma.wait()
            out_ref[pl.ds(other, sz), :] = ag_r[bf][u][...].astype(jnp.float32)


def kernel(x, Wq, K_ext, V_ext, Wo):
    i = lax.axis_index("i")
    x2 = x[0]
    K = lax.dynamic_slice_in_dim(K_ext[0], i * HQ_PER, HQ_PER, axis=1)
    V = lax.dynamic_slice_in_dim(V_ext[0], i * HQ_PER, HQ_PER, axis=1)

    partial = pl.pallas_call(
        _compute_body,
        out_shape=jax.ShapeDtypeStruct((SQ, 1024), jnp.float32),
        in_specs=[pl.BlockSpec(memory_space=pltpu.VMEM)] * 5,
        out_specs=pl.BlockSpec(memory_space=pltpu.VMEM),
    )(x2, Wq, K, V, Wo)

    out = pl.pallas_call(
        _allreduce_body,
        out_shape=jax.ShapeDtypeStruct((SQ, 1024), jnp.float32),
        in_specs=[pl.BlockSpec(memory_space=pltpu.VMEM)],
        out_specs=pl.BlockSpec(memory_space=pltpu.VMEM),
        scratch_shapes=(
            sum(
                (
                    [pltpu.VMEM((_rs_rows(t), 1024), jnp.bfloat16) for t in range(4)]
                    + [pltpu.VMEM((_rs_rows(t), 1024), jnp.bfloat16) for t in range(4)]
                    + [pltpu.VMEM((_ag_rows(u), 1024), jnp.bfloat16) for u in range(4)]
                    + [pltpu.VMEM((_ag_rows(u), 1024), jnp.bfloat16) for u in range(4)]
                    for _bf in range(2)
                ),
                [],
            )
            + [pltpu.SemaphoreType.DMA((4,)) for _ in range(8)]
        ),
        compiler_params=pltpu.CompilerParams(collective_id=0),
    )(partial)

    return out[None]


# device time: 78796 ns/iter; 2.3583x vs baseline; 1.0205x over previous
import jax
import jax.numpy as jnp
from jax import lax
from jax.experimental import pallas as pl
from jax.experimental.pallas import tpu as pltpu

N_DEV = 16
SQ = 1024
SKV = 1024
HQ_PER = 8
DH = 128
SCALE = 0.08838834764831843


def _compute_body(x_ref, wq_ref, k_hbm, v_hbm, wo_ref, out_ref,
                  k_vmem, v_vmem, kv_sem):
    my = lax.axis_index("i")
    h0 = my * HQ_PER
    kcopy = pltpu.make_async_copy(
        k_hbm.at[0, :, pl.ds(h0, HQ_PER), :], k_vmem, kv_sem.at[0]
    )
    vcopy = pltpu.make_async_copy(
        v_hbm.at[0, :, pl.ds(h0, HQ_PER), :], v_vmem, kv_sem.at[1]
    )
    kcopy.start()
    vcopy.start()

    xb = x_ref[0].astype(jnp.bfloat16)
    wqb = wq_ref[...].astype(jnp.bfloat16)
    q = (
        lax.dot_general(
            xb, wqb, (((1,), (0,)), ((), ())),
            preferred_element_type=jnp.float32,
        )
        * SCALE
    ).astype(jnp.bfloat16)

    qi = lax.broadcasted_iota(jnp.int32, (SQ, SKV), 0)
    ki = lax.broadcasted_iota(jnp.int32, (SQ, SKV), 1)
    mask = (jnp.abs(qi - ki) <= 128) | (ki < 32) | (qi < 32)
    bias = jnp.where(mask, 0.0, -1e9).astype(jnp.float32)

    kcopy.wait()
    vcopy.wait()

    ctx_parts = []
    for h in range(HQ_PER):
        qh = q[:, h * DH:(h + 1) * DH]
        kh = k_vmem[:, h, :].astype(jnp.bfloat16)
        vh = v_vmem[:, h, :].astype(jnp.bfloat16)
        s = lax.dot_general(
            qh, kh, (((1,), (1,)), ((), ())), preferred_element_type=jnp.float32
        ) + bias
        w = jnp.exp(s)
        denom = jnp.sum(w, axis=1, keepdims=True)
        ctxh = lax.dot_general(
            w.astype(jnp.bfloat16), vh, (((1,), (0,)), ((), ())),
            preferred_element_type=jnp.float32,
        ) / denom
        ctx_parts.append(ctxh)
    ctx = jnp.concatenate(ctx_parts, axis=1).astype(jnp.bfloat16)
    wob = wo_ref[...].astype(jnp.bfloat16)
    out_ref[...] = lax.dot_general(
        ctx, wob, (((1,), (0,)), ((), ())), preferred_element_type=jnp.float32
    )


def _hypercube_partners():
    my = lax.axis_index("i")
    z = my // 4
    j = my % 4
    b = j // 2
    a = (j % 2) ^ b
    h = [a, b, z % 2, z // 2]

    def to_logical(hb):
        jj = (hb[0] ^ hb[1]) + 2 * hb[1]
        return 4 * (2 * hb[3] + hb[2]) + jj

    partners = []
    for k in range(4):
        hb = list(h)
        hb[k] = 1 - hb[k]
        partners.append(to_logical(hb))
    return h, partners


ORDERS = ((0, 1, 2, 3), (1, 2, 0, 3))
HALF_ROWS = SQ // 2


def _rs_rows(t):
    return HALF_ROWS >> (t + 1)


def _ag_rows(u):
    return 32 << u


def _allreduce_body(p_ref, out_ref, *s):
    del p_ref
    rs_s = (s[0:4], s[16:20])
    rs_r = (s[4:8], s[20:24])
    ag_s = (s[8:12], s[24:28])
    ag_r = (s[12:16], s[28:32])
    sems = s[32:40]
    rs_ss = (sems[0], sems[4])
    rs_rs = (sems[1], sems[5])
    ag_ss = (sems[2], sems[6])
    ag_rs = (sems[3], sems[7])

    h, partners = _hypercube_partners()

    barrier_sem = pltpu.get_barrier_semaphore()
    for d in range(4):
        pl.semaphore_signal(
            barrier_sem, inc=1,
            device_id=(partners[d],), device_id_type=pl.DeviceIdType.MESH,
        )
    pl.semaphore_wait(barrier_sem, 4)

    r = [0, HALF_ROWS]
    size = [HALF_ROWS, HALF_ROWS]

    for t in range(4):
        started = []
        for bf in range(2):
            d = ORDERS[bf][t]
            half = size[bf] // 2
            keep = r[bf] + h[d] * half
            send = r[bf] + (1 - h[d]) * half
            rs_s[bf][t][...] = out_ref[pl.ds(send, half), :].astype(jnp.bfloat16)
            rdma = pltpu.make_async_remote_copy(
                src_ref=rs_s[bf][t],
                dst_ref=rs_r[bf][t],
                send_sem=rs_ss[bf].at[t],
                recv_sem=rs_rs[bf].at[t],
                device_id=(partners[d],),
                device_id_type=pl.DeviceIdType.MESH,
            )
            rdma.start()
            started.append((rdma, keep, half))
            r[bf] = keep
            size[bf] = half
        for bf in range(2):
            rdma, keep, half = started[bf]
            rdma.wait()
            out_ref[pl.ds(keep, half), :] = (
                out_ref[pl.ds(keep, half), :]
                + rs_r[bf][t][...].astype(jnp.float32)
            )

    for u in range(4):
        started = []
        for bf in range(2):
            d = ORDERS[bf][3 - u]
            sz = size[bf]
            base = r[bf] - h[d] * sz
            other = base + (1 - h[d]) * sz
            ag_s[bf][u][...] = out_ref[pl.ds(r[bf], sz), :].astype(jnp.bfloat16)
            rdma = pltpu.make_async_remote_copy(
                src_ref=ag_s[bf][u],
                dst_ref=ag_r[bf][u],
                send_sem=ag_ss[bf].at[u],
                recv_sem=ag_rs[bf].at[u],
                device_id=(partners[d],),
                device_id_type=pl.DeviceIdType.MESH,
            )
            rdma.start()
            started.append((rdma, other, sz))
            r[bf] = base
            size[bf] = sz * 2
        for bf in range(2):
            rdma, other, sz = started[bf]
            rdma.wait()
            out_ref[pl.ds(other, sz), :] = ag_r[bf][u][...].astype(jnp.float32)


def kernel(x, Wq, K_ext, V_ext, Wo):
    partial = pl.pallas_call(
        _compute_body,
        out_shape=jax.ShapeDtypeStruct((SQ, 1024), jnp.float32),
        in_specs=[
            pl.BlockSpec(memory_space=pltpu.VMEM),
            pl.BlockSpec(memory_space=pltpu.VMEM),
            pl.BlockSpec(memory_space=pl.ANY),
            pl.BlockSpec(memory_space=pl.ANY),
            pl.BlockSpec(memory_space=pltpu.VMEM),
        ],
        out_specs=pl.BlockSpec(memory_space=pltpu.VMEM),
        scratch_shapes=[
            pltpu.VMEM((SKV, HQ_PER, DH), jnp.float32),
            pltpu.VMEM((SKV, HQ_PER, DH), jnp.float32),
            pltpu.SemaphoreType.DMA((2,)),
        ],
    )(x, Wq, K_ext, V_ext, Wo)

    out = pl.pallas_call(
        _allreduce_body,
        out_shape=jax.ShapeDtypeStruct((SQ, 1024), jnp.float32),
        in_specs=[pl.BlockSpec(memory_space=pltpu.VMEM)],
        out_specs=pl.BlockSpec(memory_space=pltpu.VMEM),
        input_output_aliases={0: 0},
        scratch_shapes=(
            sum(
                (
                    [pltpu.VMEM((_rs_rows(t), 1024), jnp.bfloat16) for t in range(4)]
                    + [pltpu.VMEM((_rs_rows(t), 1024), jnp.bfloat16) for t in range(4)]
                    + [pltpu.VMEM((_ag_rows(u), 1024), jnp.bfloat16) for u in range(4)]
                    + [pltpu.VMEM((_ag_rows(u), 1024), jnp.bfloat16) for u in range(4)]
                    for _bf in range(2)
                ),
                [],
            )
            + [pltpu.SemaphoreType.DMA((4,)) for _ in range(8)]
        ),
        compiler_params=pltpu.CompilerParams(collective_id=0),
    )(partial)

    return out[None]


# device time: 74342 ns/iter; 2.4996x vs baseline; 1.0599x over previous
import jax
import jax.numpy as jnp
from jax import lax
from jax.experimental import pallas as pl
from jax.experimental.pallas import tpu as pltpu

N_DEV = 16
SQ = 1024
SKV = 1024
HQ_PER = 8
DH = 128
SCALE = 0.08838834764831843


def _fused_body(x_ref, wq_ref, k_hbm, v_hbm, wo_ref, out_ref,
                k_vmem, v_vmem, kv_sem, q_vmem, *ar_scratch):
    my = lax.axis_index("i")
    h0 = my * HQ_PER

    hbits, partners = _hypercube_partners()
    barrier_sem = pltpu.get_barrier_semaphore()
    for d in range(4):
        pl.semaphore_signal(
            barrier_sem, inc=1,
            device_id=(partners[d],), device_id_type=pl.DeviceIdType.MESH,
        )
    pl.semaphore_wait(barrier_sem, 4)
    kcopy = pltpu.make_async_copy(
        k_hbm.at[0, :, pl.ds(h0, HQ_PER), :], k_vmem, kv_sem.at[0]
    )
    vcopy = pltpu.make_async_copy(
        v_hbm.at[0, :, pl.ds(h0, HQ_PER), :], v_vmem, kv_sem.at[1]
    )
    kcopy.start()
    vcopy.start()

    xb = x_ref[0].astype(jnp.bfloat16)
    wqb = wq_ref[...].astype(jnp.bfloat16)
    q_vmem[...] = (
        lax.dot_general(
            xb, wqb, (((1,), (0,)), ((), ())),
            preferred_element_type=jnp.float32,
        )
        * SCALE
    ).astype(jnp.bfloat16)

    kcopy.wait()
    vcopy.wait()
    khs = [k_vmem[:, hh, :].astype(jnp.bfloat16) for hh in range(HQ_PER)]
    vhs = [v_vmem[:, hh, :].astype(jnp.bfloat16) for hh in range(HQ_PER)]
    wob = wo_ref[...].astype(jnp.bfloat16)

    CH = SQ // 4

    def compute_chunk(off):
        qi = off + lax.broadcasted_iota(jnp.int32, (CH, SKV), 0)
        ki = lax.broadcasted_iota(jnp.int32, (CH, SKV), 1)
        mask = (jnp.abs(qi - ki) <= 128) | (ki < 32) | (qi < 32)
        bias = jnp.where(mask, 0.0, -1e9).astype(jnp.float32)
        ctx_parts = []
        for hh in range(HQ_PER):
            qh = q_vmem[pl.ds(off, CH), hh * DH:(hh + 1) * DH]
            s = lax.dot_general(
                qh, khs[hh], (((1,), (1,)), ((), ())),
                preferred_element_type=jnp.float32,
            ) + bias
            w = jnp.exp(s)
            denom = jnp.sum(w, axis=1, keepdims=True)
            ctx_parts.append(
                lax.dot_general(
                    w.astype(jnp.bfloat16), vhs[hh], (((1,), (0,)), ((), ())),
                    preferred_element_type=jnp.float32,
                ) / denom
            )
        ctx = jnp.concatenate(ctx_parts, axis=1).astype(jnp.bfloat16)
        out_ref[pl.ds(off, CH), :] = lax.dot_general(
            ctx, wob, (((1,), (0,)), ((), ())),
            preferred_element_type=jnp.float32,
        )

    send_a = (1 - hbits[ORDERS[0][0]]) * CH
    send_b = HALF_ROWS + (1 - hbits[ORDERS[1][0]]) * CH
    keep_a = hbits[ORDERS[0][0]] * CH
    keep_b = HALF_ROWS + hbits[ORDERS[1][0]] * CH
    compute_chunk(send_a)
    compute_chunk(send_b)
    started, r, size = _ar_start_stage0(out_ref, hbits, partners, ar_scratch)
    compute_chunk(keep_a)
    compute_chunk(keep_b)
    _ar_rest(out_ref, hbits, partners, ar_scratch, started, r, size)


def _hypercube_partners():
    my = lax.axis_index("i")
    z = my // 4
    j = my % 4
    b = j // 2
    a = (j % 2) ^ b
    h = [a, b, z % 2, z // 2]

    def to_logical(hb):
        jj = (hb[0] ^ hb[1]) + 2 * hb[1]
        return 4 * (2 * hb[3] + hb[2]) + jj

    partners = []
    for k in range(4):
        hb = list(h)
        hb[k] = 1 - hb[k]
        partners.append(to_logical(hb))
    return h, partners


ORDERS = ((0, 1, 2, 3), (1, 2, 0, 3))
HALF_ROWS = SQ // 2


def _rs_rows(t):
    return HALF_ROWS >> (t + 1)


def _ag_rows(u):
    return 32 << u


def _ar_unpack(s):
    rs_s = (s[0:4], s[16:20])
    rs_r = (s[4:8], s[20:24])
    ag_s = (s[8:12], s[24:28])
    ag_r = (s[12:16], s[28:32])
    sems = s[32:40]
    rs_ss = (sems[0], sems[4])
    rs_rs = (sems[1], sems[5])
    ag_ss = (sems[2], sems[6])
    ag_rs = (sems[3], sems[7])
    return rs_s, rs_r, ag_s, ag_r, rs_ss, rs_rs, ag_ss, ag_rs


def _rs_step_start(out_ref, hbits, partners, rs_s, rs_r, rs_ss, rs_rs,
                   r, size, t):
    geo = []
    for bf in range(2):
        d = ORDERS[bf][t]
        half = size[bf] // 2
        keep = r[bf] + hbits[d] * half
        send = r[bf] + (1 - hbits[d]) * half
        geo.append((d, half // 2, keep, send))
        r[bf] = keep
        size[bf] = half
    started = []
    for c in range(2):
        for bf in range(2):
            d, sub, keep, send = geo[bf]
            rs_s[bf][t][pl.ds(c * sub, sub), :] = out_ref[
                pl.ds(send + c * sub, sub), :
            ].astype(jnp.bfloat16)
            rdma = pltpu.make_async_remote_copy(
                src_ref=rs_s[bf][t].at[pl.ds(c * sub, sub)],
                dst_ref=rs_r[bf][t].at[pl.ds(c * sub, sub)],
                send_sem=rs_ss[bf].at[t, c],
                recv_sem=rs_rs[bf].at[t, c],
                device_id=(partners[d],),
                device_id_type=pl.DeviceIdType.MESH,
            )
            rdma.start()
            started.append((rdma, bf, c, keep + c * sub, sub))
    return started


def _rs_step_finish(out_ref, rs_r, started, t):
    for rdma, bf, c, off, sub in started:
        rdma.wait()
        out_ref[pl.ds(off, sub), :] = (
            out_ref[pl.ds(off, sub), :]
            + rs_r[bf][t][pl.ds(c * sub, sub), :].astype(jnp.float32)
        )


def _ar_start_stage0(out_ref, hbits, partners, s):
    rs_s, rs_r, _, _, rs_ss, rs_rs, _, _ = _ar_unpack(s)
    r = [0, HALF_ROWS]
    size = [HALF_ROWS, HALF_ROWS]
    started = _rs_step_start(
        out_ref, hbits, partners, rs_s, rs_r, rs_ss, rs_rs, r, size, 0
    )
    return started, r, size


def _ar_rest(out_ref, hbits, partners, s, started, r, size):
    rs_s, rs_r, ag_s, ag_r, rs_ss, rs_rs, ag_ss, ag_rs = _ar_unpack(s)

    _rs_step_finish(out_ref, rs_r, started, 0)
    for t in range(1, 4):
        started = _rs_step_start(
            out_ref, hbits, partners, rs_s, rs_r, rs_ss, rs_rs, r, size, t
        )
        _rs_step_finish(out_ref, rs_r, started, t)

    for u in range(4):
        geo = []
        for bf in range(2):
            d = ORDERS[bf][3 - u]
            sz = size[bf]
            base = r[bf] - hbits[d] * sz
            other = base + (1 - hbits[d]) * sz
            geo.append((d, sz // 2, r[bf], other))
            r[bf] = base
            size[bf] = sz * 2
        started = []
        for c in range(2):
            for bf in range(2):
                d, sub, mine, other = geo[bf]
                ag_s[bf][u][pl.ds(c * sub, sub), :] = out_ref[
                    pl.ds(mine + c * sub, sub), :
                ].astype(jnp.bfloat16)
                rdma = pltpu.make_async_remote_copy(
                    src_ref=ag_s[bf][u].at[pl.ds(c * sub, sub)],
                    dst_ref=ag_r[bf][u].at[pl.ds(c * sub, sub)],
                    send_sem=ag_ss[bf].at[u, c],
                    recv_sem=ag_rs[bf].at[u, c],
                    device_id=(partners[d],),
                    device_id_type=pl.DeviceIdType.MESH,
                )
                rdma.start()
                started.append((rdma, bf, c, other + c * sub, sub))
        for rdma, bf, c, off, sub in started:
            rdma.wait()
            out_ref[pl.ds(off, sub), :] = ag_r[bf][u][
                pl.ds(c * sub, sub), :
            ].astype(jnp.float32)


def kernel(x, Wq, K_ext, V_ext, Wo):
    out = pl.pallas_call(
        _fused_body,
        out_shape=jax.ShapeDtypeStruct((SQ, 1024), jnp.float32),
        in_specs=[
            pl.BlockSpec(memory_space=pltpu.VMEM),
            pl.BlockSpec(memory_space=pltpu.VMEM),
            pl.BlockSpec(memory_space=pl.ANY),
            pl.BlockSpec(memory_space=pl.ANY),
            pl.BlockSpec(memory_space=pltpu.VMEM),
        ],
        out_specs=pl.BlockSpec(memory_space=pltpu.VMEM),
        scratch_shapes=(
            [
                pltpu.VMEM((SKV, HQ_PER, DH), jnp.float32),
                pltpu.VMEM((SKV, HQ_PER, DH), jnp.float32),
                pltpu.SemaphoreType.DMA((2,)),
                pltpu.VMEM((SQ, HQ_PER * DH), jnp.bfloat16),
            ]
            + sum(
                (
                    [pltpu.VMEM((_rs_rows(t), 1024), jnp.bfloat16) for t in range(4)]
                    + [pltpu.VMEM((_rs_rows(t), 1024), jnp.bfloat16) for t in range(4)]
                    + [pltpu.VMEM((_ag_rows(u), 1024), jnp.bfloat16) for u in range(4)]
                    + [pltpu.VMEM((_ag_rows(u), 1024), jnp.bfloat16) for u in range(4)]
                    for _bf in range(2)
                ),
                [],
            )
            + [pltpu.SemaphoreType.DMA((4, 2)) for _ in range(8)]
        ),
        compiler_params=pltpu.CompilerParams(collective_id=0),
    )(x, Wq, K_ext, V_ext, Wo)

    return out[None]
